# baseline (speedup 1.0000x reference)
import functools

import jax
import jax.numpy as jnp
from jax import lax
from jax.experimental import pallas as pl
from jax.experimental.pallas import tpu as pltpu
from jax.experimental.pallas import tpu_sc as plsc

N_CLS = 1000
N_CTX = 16
DIM = 512
SEQ = 77
SUF = SEQ - 1 - N_CTX
RB = 32


def kernel(ctx, token_prefix, token_suffix):
    info = plsc.get_sparse_core_info()
    nc, ns = info.num_cores, info.num_subcores
    nw = nc * ns
    tail_rows = N_CLS - (nw - 1) * RB

    mesh = plsc.VectorSubcoreMesh(core_axis_name="c", subcore_axis_name="s")

    @functools.partial(
        pl.kernel,
        mesh=mesh,
        out_type=jax.ShapeDtypeStruct((SEQ, N_CLS, DIM), jnp.float32),
        scratch_types=[
            pltpu.VMEM((N_CTX, DIM), jnp.float32),
            pltpu.VMEM((N_CTX, 8, DIM), jnp.float32),
            pltpu.VMEM((2, RB, DIM), jnp.float32),
            pltpu.VMEM((RB, DIM), jnp.float32),
            pltpu.SemaphoreType.DMA,
            pltpu.SemaphoreType.DMA,
            pltpu.SemaphoreType.DMA,
            pltpu.SemaphoreType.DMA,
        ],
    )
    def prompt_assemble(ctx_hbm, pre_hbm, suf_hbm, out_hbm, ctx_buf, brd_buf,
                        sbuf, pbuf, sem_in, sem_out, sem_pre, sem_ctx):
        cid = lax.axis_index("c")
        sid = lax.axis_index("s")
        wid = sid * nc + cid
        r0 = wid * RB

        def do_rows(nr):
            def s_copy(s):
                return pltpu.make_async_copy(suf_hbm.at[s, pl.ds(r0, nr)],
                                             out_hbm.at[17 + s,
                                                        pl.ds(r0, nr)],
                                             sem_in)

            @pl.loop(0, SUF)
            def _(s):
                s_copy(s).start()

            pg = pltpu.make_async_copy(pre_hbm.at[pl.ds(r0, nr)],
                                       pbuf.at[pl.ds(0, nr)], sem_pre)
            pg.start()
            pg.wait()
            ps = pltpu.make_async_copy(pbuf.at[pl.ds(0, nr)],
                                       out_hbm.at[0, pl.ds(r0, nr)], sem_pre)
            ps.start()

            pltpu.sync_copy(ctx_hbm, ctx_buf)

            @pl.loop(0, N_CTX)
            def _(j):
                for l in range(0, DIM, 16):
                    v = ctx_buf[j, pl.ds(l, 16)]
                    for k in range(8):
                        brd_buf[j, k, pl.ds(l, 16)] = v

            ctx_copies = [
                pltpu.make_async_copy(
                    brd_buf.at[j],
                    out_hbm.at[1 + j, pl.ds(r0 + 8 * k, 8)], sem_ctx)
                for j in range(N_CTX) for k in range(nr // 8)
            ]
            for h in ctx_copies:
                h.start()

            ps.wait()
            for h in ctx_copies:
                h.wait()

            @pl.loop(0, SUF)
            def _(s):
                s_copy(s).wait()

        @pl.when(wid < nw - 1)
        def _():
            do_rows(RB)

        @pl.when(wid == nw - 1)
        def _():
            do_rows(tail_rows)

    pre2 = token_prefix.reshape(N_CLS, DIM)
    suf_t = jnp.transpose(token_suffix, (1, 0, 2))
    out_t = prompt_assemble(ctx, pre2, suf_t)
    return jnp.transpose(out_t, (1, 0, 2))


# speedup vs baseline: 25.4724x; 25.4724x over previous
import functools

import jax
import jax.numpy as jnp
from jax import lax
from jax.experimental import pallas as pl
from jax.experimental.pallas import tpu as pltpu
from jax.experimental.pallas import tpu_sc as plsc

N_CLS = 1000
N_CTX = 16
DIM = 512
SEQ = 77
SUF = SEQ - 1 - N_CTX
RB = 32


def kernel(ctx, token_prefix, token_suffix):
    info = plsc.get_sparse_core_info()
    nc, ns = info.num_cores, info.num_subcores
    nw = nc * ns
    tail_rows = N_CLS - (nw - 1) * RB

    mesh = plsc.VectorSubcoreMesh(core_axis_name="c", subcore_axis_name="s")

    @functools.partial(
        pl.kernel,
        mesh=mesh,
        out_type=jax.ShapeDtypeStruct((SEQ, N_CLS, DIM), jnp.float32),
        scratch_types=[
            pltpu.VMEM((N_CTX, DIM), jnp.float32),
            pltpu.VMEM((N_CTX, 8, DIM), jnp.float32),
            pltpu.VMEM((2, RB, DIM), jnp.float32),
            pltpu.VMEM((RB, DIM), jnp.float32),
            pltpu.SemaphoreType.DMA,
            pltpu.SemaphoreType.DMA,
            pltpu.SemaphoreType.DMA,
            pltpu.SemaphoreType.DMA,
        ],
    )
    def prompt_assemble(ctx_hbm, pre_hbm, suf_hbm, out_hbm, ctx_buf, brd_buf,
                        sbuf, pbuf, sem_in, sem_out, sem_pre, sem_ctx):
        cid = lax.axis_index("c")
        sid = lax.axis_index("s")
        wid = sid * nc + cid
        r0 = wid * RB

        def do_rows(nr):
            def s_gather(s, slot):
                return pltpu.make_async_copy(suf_hbm.at[s, pl.ds(r0, nr)],
                                             sbuf.at[slot, pl.ds(0, nr)],
                                             sem_in)

            def s_scatter(s, slot):
                return pltpu.make_async_copy(sbuf.at[slot, pl.ds(0, nr)],
                                             out_hbm.at[17 + s,
                                                        pl.ds(r0, nr)],
                                             sem_out)

            s_gather(0, 0).start()
            pg = pltpu.make_async_copy(pre_hbm.at[pl.ds(r0, nr)],
                                       pbuf.at[pl.ds(0, nr)], sem_pre)
            pg.start()
            cg = pltpu.make_async_copy(ctx_hbm, ctx_buf, sem_ctx)
            cg.start()

            pg.wait()
            ps = pltpu.make_async_copy(pbuf.at[pl.ds(0, nr)],
                                       out_hbm.at[0, pl.ds(r0, nr)], sem_pre)
            ps.start()

            cg.wait()

            @pl.loop(0, N_CTX)
            def _(j):
                for l in range(0, DIM, 16):
                    v = ctx_buf[j, pl.ds(l, 16)]
                    for k in range(8):
                        brd_buf[j, k, pl.ds(l, 16)] = v

            ctx_copies = [
                pltpu.make_async_copy(
                    brd_buf.at[j],
                    out_hbm.at[1 + j, pl.ds(r0 + 8 * k, 8)], sem_ctx)
                for j in range(N_CTX) for k in range(nr // 8)
            ]
            for h in ctx_copies:
                h.start()

            @pl.loop(0, SUF)
            def _(s):
                slot = lax.rem(s, 2)
                s_gather(s, slot).wait()

                @pl.when(s >= 1)
                def _():
                    s_scatter(s - 1, 1 - slot).wait()

                @pl.when(s + 1 < SUF)
                def _():
                    s_gather(s + 1, 1 - slot).start()

                s_scatter(s, slot).start()

            s_scatter(SUF - 1, lax.rem(SUF - 1, 2)).wait()
            ps.wait()
            for h in ctx_copies:
                h.wait()

        @pl.when(wid < nw - 1)
        def _():
            do_rows(RB)

        @pl.when(wid == nw - 1)
        def _():
            do_rows(tail_rows)

    pre2 = token_prefix.reshape(N_CLS, DIM)
    suf_t = jnp.transpose(token_suffix, (1, 0, 2))
    out_t = prompt_assemble(ctx, pre2, suf_t)
    return jnp.transpose(out_t, (1, 0, 2))


# speedup vs baseline: 29.7342x; 1.1673x over previous
import functools

import jax
import jax.numpy as jnp
from jax import lax
from jax.experimental import pallas as pl
from jax.experimental.pallas import tpu as pltpu
from jax.experimental.pallas import tpu_sc as plsc

N_CLS = 1000
N_CTX = 16
DIM = 512
SEQ = 77
SUF = SEQ - 1 - N_CTX
RB = 32


def kernel(ctx, token_prefix, token_suffix):
    info = plsc.get_sparse_core_info()
    nc, ns = info.num_cores, info.num_subcores
    nw = nc * ns
    tail_rows = N_CLS - (nw - 1) * RB

    mesh = plsc.VectorSubcoreMesh(core_axis_name="c", subcore_axis_name="s")

    @functools.partial(
        pl.kernel,
        mesh=mesh,
        out_type=jax.ShapeDtypeStruct((SEQ, N_CLS, DIM), jnp.float32),
        scratch_types=[
            pltpu.VMEM((N_CTX, DIM), jnp.float32),
            pltpu.VMEM((N_CTX, 8, DIM), jnp.float32),
            pltpu.VMEM((3, RB, DIM), jnp.float32),
            pltpu.SemaphoreType.DMA,
            pltpu.SemaphoreType.DMA,
            pltpu.SemaphoreType.DMA,
            pltpu.SemaphoreType.DMA,
        ],
    )
    def prompt_assemble(ctx_hbm, pre_hbm, suf_hbm, out_hbm, ctx_buf, brd_buf,
                        sbuf, sem_in, sem_out, sem_pre, sem_ctx):
        cid = lax.axis_index("c")
        sid = lax.axis_index("s")
        wid = sid * nc + cid
        r0 = wid * RB

        def do_rows(nr):
            def s_gather(s, slot):
                return pltpu.make_async_copy(suf_hbm.at[s, pl.ds(r0, nr)],
                                             sbuf.at[slot, pl.ds(0, nr)],
                                             sem_in)

            def s_scatter(s, slot):
                return pltpu.make_async_copy(sbuf.at[slot, pl.ds(0, nr)],
                                             out_hbm.at[17 + s,
                                                        pl.ds(r0, nr)],
                                             sem_out)

            s_gather(0, 0).start()
            s_gather(1, 1).start()
            pg = pltpu.make_async_copy(pre_hbm.at[pl.ds(r0, nr)],
                                       sbuf.at[2, pl.ds(0, nr)], sem_pre)
            pg.start()
            cg = pltpu.make_async_copy(ctx_hbm, ctx_buf, sem_ctx)
            cg.start()

            pg.wait()
            ps = pltpu.make_async_copy(sbuf.at[2, pl.ds(0, nr)],
                                       out_hbm.at[0, pl.ds(r0, nr)], sem_pre)
            ps.start()

            cg.wait()

            @pl.loop(0, N_CTX)
            def _(j):
                for l in range(0, DIM, 16):
                    v = ctx_buf[j, pl.ds(l, 16)]
                    for k in range(8):
                        brd_buf[j, k, pl.ds(l, 16)] = v

            ctx_copies = [
                pltpu.make_async_copy(
                    brd_buf.at[j],
                    out_hbm.at[1 + j, pl.ds(r0 + 8 * k, 8)], sem_ctx)
                for j in range(N_CTX) for k in range(nr // 8)
            ]
            for h in ctx_copies:
                h.start()

            ps.wait()

            @pl.loop(0, SUF)
            def _(s):
                slot = lax.rem(s, 3)
                s_gather(s, slot).wait()
                s_scatter(s, slot).start()

                @pl.when(s >= 1)
                def _():
                    s_scatter(s - 1, lax.rem(s + 2, 3)).wait()

                @pl.when(s + 2 < SUF)
                def _():
                    s_gather(s + 2, lax.rem(s + 2, 3)).start()

            s_scatter(SUF - 1, lax.rem(SUF - 1, 3)).wait()
            for h in ctx_copies:
                h.wait()

        @pl.when(wid < nw - 1)
        def _():
            do_rows(RB)

        @pl.when(wid == nw - 1)
        def _():
            do_rows(tail_rows)

    pre2 = token_prefix.reshape(N_CLS, DIM)
    suf_t = jnp.transpose(token_suffix, (1, 0, 2))
    out_t = prompt_assemble(ctx, pre2, suf_t)
    return jnp.transpose(out_t, (1, 0, 2))


# speedup vs baseline: 29.7509x; 1.0006x over previous
import functools

import jax
import jax.numpy as jnp
from jax import lax
from jax.experimental import pallas as pl
from jax.experimental.pallas import tpu as pltpu
from jax.experimental.pallas import tpu_sc as plsc

N_CLS = 1000
N_CTX = 16
DIM = 512
SEQ = 77
SUF = SEQ - 1 - N_CTX
RB = 32
DEPTH = 4
BRD = 12
SPLIT = 30


def kernel(ctx, token_prefix, token_suffix):
    info = plsc.get_sparse_core_info()
    nc, ns = info.num_cores, info.num_subcores
    nw = nc * ns
    tail_rows = N_CLS - (nw - 1) * RB

    mesh = plsc.VectorSubcoreMesh(core_axis_name="c", subcore_axis_name="s")

    @functools.partial(
        pl.kernel,
        mesh=mesh,
        out_type=jax.ShapeDtypeStruct((SEQ, N_CLS, DIM), jnp.float32),
        scratch_types=[
            pltpu.VMEM((N_CTX, DIM), jnp.float32),
            pltpu.VMEM((BRD, 8, DIM), jnp.float32),
            pltpu.VMEM((DEPTH, RB, DIM), jnp.float32),
            pltpu.SemaphoreType.DMA,
            pltpu.SemaphoreType.DMA,
            pltpu.SemaphoreType.DMA,
            pltpu.SemaphoreType.DMA,
        ],
    )
    def prompt_assemble(ctx_hbm, pre_hbm, suf_hbm, out_hbm, ctx_buf, brd_buf,
                        sbuf, sem_in, sem_out, sem_pre, sem_ctx):
        cid = lax.axis_index("c")
        sid = lax.axis_index("s")
        wid = sid * nc + cid
        r0 = wid * RB

        def do_rows(nr):
            def s_gather(s, slot):
                return pltpu.make_async_copy(suf_hbm.at[s, pl.ds(r0, nr)],
                                             sbuf.at[slot, pl.ds(0, nr)],
                                             sem_in)

            def s_scatter(s, slot):
                return pltpu.make_async_copy(sbuf.at[slot, pl.ds(0, nr)],
                                             out_hbm.at[17 + s,
                                                        pl.ds(r0, nr)],
                                             sem_out)

            def ctx_copy(j, k):
                return pltpu.make_async_copy(
                    brd_buf.at[j % BRD],
                    out_hbm.at[1 + j, pl.ds(r0 + 8 * k, 8)], sem_ctx)

            def fill_brd(j0, nplanes):
                @pl.loop(0, nplanes)
                def _(j):
                    for l in range(0, DIM, 16):
                        v = ctx_buf[j0 + j, pl.ds(l, 16)]
                        for k in range(8):
                            brd_buf[j, k, pl.ds(l, 16)] = v

            for s in range(DEPTH - 1):
                s_gather(s, s).start()
            pg = pltpu.make_async_copy(pre_hbm.at[pl.ds(r0, nr)],
                                       sbuf.at[DEPTH - 1, pl.ds(0, nr)],
                                       sem_pre)
            pg.start()
            cg = pltpu.make_async_copy(ctx_hbm, ctx_buf, sem_ctx)
            cg.start()

            pg.wait()
            ps = pltpu.make_async_copy(sbuf.at[DEPTH - 1, pl.ds(0, nr)],
                                       out_hbm.at[0, pl.ds(r0, nr)], sem_pre)
            ps.start()

            cg.wait()
            fill_brd(0, BRD)
            pass1 = [
                ctx_copy(j, k) for j in range(BRD) for k in range(nr // 8)
            ]
            for h in pass1:
                h.start()

            ps.wait()

            def ring_body(s):
                slot = lax.rem(s, DEPTH)
                s_gather(s, slot).wait()
                s_scatter(s, slot).start()

                @pl.when(s >= 1)
                def _():
                    s_scatter(s - 1, lax.rem(s + DEPTH - 1, DEPTH)).wait()

                @pl.when(s + DEPTH - 1 < SUF)
                def _():
                    s_gather(s + DEPTH - 1,
                             lax.rem(s + DEPTH - 1, DEPTH)).start()

            @pl.loop(0, SPLIT)
            def _(s):
                ring_body(s)

            for h in pass1:
                h.wait()
            fill_brd(BRD, N_CTX - BRD)
            pass2 = [
                ctx_copy(j, k) for j in range(BRD, N_CTX)
                for k in range(nr // 8)
            ]
            for h in pass2:
                h.start()

            @pl.loop(SPLIT, SUF)
            def _(s):
                ring_body(s)

            s_scatter(SUF - 1, lax.rem(SUF - 1, DEPTH)).wait()
            for h in pass2:
                h.wait()

        @pl.when(wid < nw - 1)
        def _():
            do_rows(RB)

        @pl.when(wid == nw - 1)
        def _():
            do_rows(tail_rows)

    pre2 = token_prefix.reshape(N_CLS, DIM)
    suf_t = jnp.transpose(token_suffix, (1, 0, 2))
    out_t = prompt_assemble(ctx, pre2, suf_t)
    return jnp.transpose(out_t, (1, 0, 2))


# speedup vs baseline: 29.8142x; 1.0021x over previous
import functools

import jax
import jax.numpy as jnp
from jax import lax
from jax.experimental import pallas as pl
from jax.experimental.pallas import tpu as pltpu
from jax.experimental.pallas import tpu_sc as plsc

N_CLS = 1000
N_CTX = 16
DIM = 512
SEQ = 77
SUF = SEQ - 1 - N_CTX
RB = 32
DEPTH = 3
BRD = 16
SPLIT = 30


def kernel(ctx, token_prefix, token_suffix):
    info = plsc.get_sparse_core_info()
    nc, ns = info.num_cores, info.num_subcores
    nw = nc * ns
    tail_rows = N_CLS - (nw - 1) * RB

    mesh = plsc.VectorSubcoreMesh(core_axis_name="c", subcore_axis_name="s")

    @functools.partial(
        pl.kernel,
        mesh=mesh,
        out_type=jax.ShapeDtypeStruct((SEQ, N_CLS, DIM), jnp.float32),
        scratch_types=[
            pltpu.VMEM((N_CTX, DIM), jnp.float32),
            pltpu.VMEM((BRD, 8, DIM), jnp.float32),
            pltpu.VMEM((DEPTH, RB, DIM), jnp.float32),
            pltpu.SemaphoreType.DMA,
            pltpu.SemaphoreType.DMA,
            pltpu.SemaphoreType.DMA,
            pltpu.SemaphoreType.DMA,
        ],
    )
    def prompt_assemble(ctx_hbm, pre_hbm, suf_hbm, out_hbm, ctx_buf, brd_buf,
                        sbuf, sem_in, sem_out, sem_pre, sem_ctx):
        cid = lax.axis_index("c")
        sid = lax.axis_index("s")
        wid = sid * nc + cid
        r0 = wid * RB

        def do_rows(nr):
            def s_gather(s, slot):
                return pltpu.make_async_copy(suf_hbm.at[s, pl.ds(r0, nr)],
                                             sbuf.at[slot, pl.ds(0, nr)],
                                             sem_in)

            def s_scatter(s, slot):
                return pltpu.make_async_copy(sbuf.at[slot, pl.ds(0, nr)],
                                             out_hbm.at[17 + s,
                                                        pl.ds(r0, nr)],
                                             sem_out)

            def ctx_copy(j, k):
                return pltpu.make_async_copy(
                    brd_buf.at[j % BRD],
                    out_hbm.at[1 + j, pl.ds(r0 + 8 * k, 8)], sem_ctx)

            def fill_brd(j0, nplanes):
                @pl.loop(0, nplanes)
                def _(j):
                    for l in range(0, DIM, 16):
                        v = ctx_buf[j0 + j, pl.ds(l, 16)]
                        for k in range(8):
                            brd_buf[j, k, pl.ds(l, 16)] = v

            for s in range(DEPTH - 1):
                s_gather(s, s).start()
            pg = pltpu.make_async_copy(pre_hbm.at[pl.ds(r0, nr)],
                                       sbuf.at[DEPTH - 1, pl.ds(0, nr)],
                                       sem_pre)
            pg.start()
            cg = pltpu.make_async_copy(ctx_hbm, ctx_buf, sem_ctx)
            cg.start()

            pg.wait()
            ps = pltpu.make_async_copy(sbuf.at[DEPTH - 1, pl.ds(0, nr)],
                                       out_hbm.at[0, pl.ds(r0, nr)], sem_pre)
            ps.start()

            cg.wait()
            fill_brd(0, BRD)
            pass1 = [
                ctx_copy(j, k) for j in range(BRD) for k in range(nr // 8)
            ]
            for h in pass1:
                h.start()

            ps.wait()

            def ring_body(s):
                slot = lax.rem(s, DEPTH)
                s_gather(s, slot).wait()
                s_scatter(s, slot).start()

                @pl.when(s >= 1)
                def _():
                    s_scatter(s - 1, lax.rem(s + DEPTH - 1, DEPTH)).wait()

                @pl.when(s + DEPTH - 1 < SUF)
                def _():
                    s_gather(s + DEPTH - 1,
                             lax.rem(s + DEPTH - 1, DEPTH)).start()

            @pl.loop(0, SPLIT)
            def _(s):
                ring_body(s)

            for h in pass1:
                h.wait()
            fill_brd(BRD, N_CTX - BRD)
            pass2 = [
                ctx_copy(j, k) for j in range(BRD, N_CTX)
                for k in range(nr // 8)
            ]
            for h in pass2:
                h.start()

            @pl.loop(SPLIT, SUF)
            def _(s):
                ring_body(s)

            s_scatter(SUF - 1, lax.rem(SUF - 1, DEPTH)).wait()
            for h in pass2:
                h.wait()

        @pl.when(wid < nw - 1)
        def _():
            do_rows(RB)

        @pl.when(wid == nw - 1)
        def _():
            do_rows(tail_rows)

    pre2 = token_prefix.reshape(N_CLS, DIM)
    suf_t = jnp.transpose(token_suffix, (1, 0, 2))
    out_t = prompt_assemble(ctx, pre2, suf_t)
    return jnp.transpose(out_t, (1, 0, 2))
